# baseline (speedup 1.0000x reference)
import jax
import jax.numpy as jnp
from jax.experimental import pallas as pl
from jax.experimental.pallas import tpu as pltpu

_EPS = 1e-09
_IGNORE = -100
_C = 19
_BH = 512
_UNROLL = 4


def _ce_body(pred_ref, ans_ref, sum_ref, cnt_ref):
    b = pl.program_id(0)
    h = pl.program_id(1)

    ans = ans_ref[0]
    tgt = jnp.clip(ans, 0, _C - 1)

    s_acc = [None] * _UNROLL
    sel_acc = [None] * _UNROLL
    for c in range(_C):
        k = c % _UNROLL
        p = pred_ref[0, c]
        s_acc[k] = p if s_acc[k] is None else s_acc[k] + p
        m = jnp.where(tgt == c, p, 0.0)
        sel_acc[k] = m if sel_acc[k] is None else sel_acc[k] + m

    s = s_acc[0]
    picked = sel_acc[0]
    for k in range(1, _UNROLL):
        s = s + s_acc[k]
        picked = picked + sel_acc[k]
    s = s + _C * _EPS

    valid = ans != _IGNORE
    contrib = jnp.where(valid, jnp.log(s) - jnp.log(picked + _EPS), 0.0)

    @pl.when((b == 0) & (h == 0))
    def _init():
        sum_ref[0, 0] = 0.0
        cnt_ref[0, 0] = 0.0

    sum_ref[0, 0] += jnp.sum(contrib)
    cnt_ref[0, 0] += jnp.sum(valid.astype(jnp.float32))


@jax.jit
def kernel(in_pred, in_ans):
    B, C, H, W = in_pred.shape
    grid = (B, H // _BH)
    sum_out, cnt_out = pl.pallas_call(
        _ce_body,
        grid=grid,
        in_specs=[
            pl.BlockSpec((1, C, _BH, W), lambda b, h: (b, 0, h, 0)),
            pl.BlockSpec((1, _BH, W), lambda b, h: (b, h, 0)),
        ],
        out_specs=[
            pl.BlockSpec(memory_space=pltpu.SMEM),
            pl.BlockSpec(memory_space=pltpu.SMEM),
        ],
        out_shape=[
            jax.ShapeDtypeStruct((1, 1), jnp.float32),
            jax.ShapeDtypeStruct((1, 1), jnp.float32),
        ],
        compiler_params=pltpu.CompilerParams(
            vmem_limit_bytes=100 * 1024 * 1024,
        ),
    )(in_pred, in_ans)
    n_valid = jnp.maximum(cnt_out[0, 0], 1.0)
    return sum_out[0, 0] / n_valid


# speedup vs baseline: 1.1204x; 1.1204x over previous
import jax
import jax.numpy as jnp
from jax.experimental import pallas as pl
from jax.experimental.pallas import tpu as pltpu

_EPS = 1e-09
_IGNORE = -100
_C = 19
_BH = 512


def _ce_body(pred_ref, ans_ref, sum_ref, cnt_ref):
    b = pl.program_id(0)
    h = pl.program_id(1)

    pred = pred_ref[0]
    ans = ans_ref[0]

    s = jnp.sum(pred, axis=0) + _C * _EPS
    tgt = jnp.clip(ans, 0, _C - 1)
    cls = jax.lax.broadcasted_iota(jnp.int32, pred.shape, 0)
    picked = jnp.sum(jnp.where(cls == tgt[None, :, :], pred, 0.0), axis=0)
    valid = ans != _IGNORE
    contrib = jnp.where(valid, jnp.log(s) - jnp.log(picked + _EPS), 0.0)

    @pl.when((b == 0) & (h == 0))
    def _init():
        sum_ref[0, 0] = 0.0
        cnt_ref[0, 0] = 0.0

    sum_ref[0, 0] += jnp.sum(contrib)
    cnt_ref[0, 0] += jnp.sum(valid.astype(jnp.float32))


@jax.jit
def kernel(in_pred, in_ans):
    B, C, H, W = in_pred.shape
    grid = (B, H // _BH)
    sum_out, cnt_out = pl.pallas_call(
        _ce_body,
        grid=grid,
        in_specs=[
            pl.BlockSpec((1, C, _BH, W), lambda b, h: (b, 0, h, 0)),
            pl.BlockSpec((1, _BH, W), lambda b, h: (b, h, 0)),
        ],
        out_specs=[
            pl.BlockSpec(memory_space=pltpu.SMEM),
            pl.BlockSpec(memory_space=pltpu.SMEM),
        ],
        out_shape=[
            jax.ShapeDtypeStruct((1, 1), jnp.float32),
            jax.ShapeDtypeStruct((1, 1), jnp.float32),
        ],
        compiler_params=pltpu.CompilerParams(
            vmem_limit_bytes=100 * 1024 * 1024,
        ),
    )(in_pred, in_ans)
    n_valid = jnp.maximum(cnt_out[0, 0], 1.0)
    return sum_out[0, 0] / n_valid
